# baseline (speedup 1.0000x reference)
import functools
import math

import numpy as np
import jax
import jax.numpy as jnp
from jax import lax
from jax.experimental import pallas as pl
from jax.experimental.pallas import tpu as pltpu
from jax.experimental.pallas import tpu_sc as plsc

N_FIELDS = 26
VOCAB = 100000
EMBED = 32
BATCH = 4096

NUM_CORES = 2
NUM_SUBCORES = 16
NW = NUM_CORES * NUM_SUBCORES
B_PER_W = BATCH // NW
LANES = 16


def _make_pe(d_model: int, max_len: int) -> np.ndarray:
    position = np.arange(max_len, dtype=np.float32)[:, None]
    div_term = np.exp(
        np.arange(0, d_model, 2, dtype=np.float32) * (-math.log(10000.0) / d_model)
    )
    pe = np.zeros((max_len, d_model), dtype=np.float32)
    pe[:, 0::2] = np.sin(position * div_term)
    pe[:, 1::2] = np.cos(position * div_term)
    return pe


_PE = _make_pe(EMBED, BATCH)

_mesh = plsc.VectorSubcoreMesh(core_axis_name="c", subcore_axis_name="s")


@functools.partial(
    pl.kernel,
    mesh=_mesh,
    out_type=jax.ShapeDtypeStruct((BATCH, N_FIELDS * EMBED), jnp.float32),
    scratch_types=[
        pltpu.VMEM((N_FIELDS, B_PER_W), jnp.int32),
        pltpu.VMEM((2, B_PER_W, EMBED), jnp.float32),
        pltpu.VMEM((B_PER_W, EMBED), jnp.float32),
        pltpu.SemaphoreType.DMA,
    ],
    compiler_params=pltpu.CompilerParams(use_tc_tiling_on_sc=False),
)
def _emb_kernel(cat_hbm, w_hbm, pe_hbm, out_hbm, idx_v, rows_v, pe_v, gsem):
    wid = lax.axis_index("s") * NUM_CORES + lax.axis_index("c")
    b0 = wid * B_PER_W

    pltpu.sync_copy(cat_hbm.at[pl.ds(0, N_FIELDS), pl.ds(b0, B_PER_W)], idx_v)
    pltpu.sync_copy(pe_hbm.at[pl.ds(b0, B_PER_W)], pe_v)

    def add_pe(buf):
        def body(j, carry):
            plsc.addupdate(buf.at[j, pl.ds(0, LANES)], pe_v[j, pl.ds(0, LANES)])
            plsc.addupdate(
                buf.at[j, pl.ds(LANES, LANES)], pe_v[j, pl.ds(LANES, LANES)]
            )
            return carry

        lax.fori_loop(0, B_PER_W, body, 0)

    cps = [None] * N_FIELDS
    cps[0] = pltpu.async_copy(w_hbm.at[0].at[idx_v.at[0]], rows_v.at[0], gsem)
    for i in range(N_FIELDS):
        if i + 1 < N_FIELDS:
            cps[i + 1] = pltpu.async_copy(
                w_hbm.at[i + 1].at[idx_v.at[i + 1]], rows_v.at[(i + 1) % 2], gsem
            )
        cps[i].wait()
        buf = rows_v.at[i % 2]
        add_pe(buf)
        pltpu.sync_copy(
            buf, out_hbm.at[pl.ds(b0, B_PER_W), pl.ds(i * EMBED, EMBED)]
        )


def kernel(cat_features, W):
    cat_t = cat_features.T.astype(jnp.int32)
    return _emb_kernel(cat_t, W, jnp.asarray(_PE))


# speedup vs baseline: 1.0081x; 1.0081x over previous
import functools
import math

import numpy as np
import jax
import jax.numpy as jnp
from jax import lax
from jax.experimental import pallas as pl
from jax.experimental.pallas import tpu as pltpu
from jax.experimental.pallas import tpu_sc as plsc

N_FIELDS = 26
VOCAB = 100000
EMBED = 32
BATCH = 4096

NUM_CORES = 2
NUM_SUBCORES = 16
NW = NUM_CORES * NUM_SUBCORES
B_PER_W = BATCH // NW
ROWS_PER_W = B_PER_W * N_FIELDS
CHUNK = 128
N_CHUNKS = ROWS_PER_W // CHUNK
LANES = 16
N_VEC = ROWS_PER_W // LANES


def _make_pe(d_model: int, max_len: int) -> np.ndarray:
    position = np.arange(max_len, dtype=np.float32)[:, None]
    div_term = np.exp(
        np.arange(0, d_model, 2, dtype=np.float32) * (-math.log(10000.0) / d_model)
    )
    pe = np.zeros((max_len, d_model), dtype=np.float32)
    pe[:, 0::2] = np.sin(position * div_term)
    pe[:, 1::2] = np.cos(position * div_term)
    return pe


_PE = _make_pe(EMBED, BATCH)

_mesh = plsc.VectorSubcoreMesh(core_axis_name="c", subcore_axis_name="s")


@functools.partial(
    pl.kernel,
    mesh=_mesh,
    out_type=jax.ShapeDtypeStruct((BATCH * N_FIELDS, EMBED), jnp.float32),
    scratch_types=[
        pltpu.VMEM((N_CHUNKS, CHUNK), jnp.int32),
        pltpu.VMEM((ROWS_PER_W, EMBED), jnp.float32),
        pltpu.VMEM((B_PER_W, EMBED), jnp.float32),
        pltpu.SemaphoreType.DMA,
    ],
    compiler_params=pltpu.CompilerParams(use_tc_tiling_on_sc=False),
)
def _emb_kernel(cat_hbm, w_hbm, pe_hbm, out_hbm, idx_v, rows_v, pe_v, sem):
    wid = lax.axis_index("s") * NUM_CORES + lax.axis_index("c")

    pltpu.sync_copy(cat_hbm.at[wid], idx_v)
    pltpu.sync_copy(pe_hbm.at[pl.ds(wid * B_PER_W, B_PER_W)], pe_v)

    def off_body(k, carry):
        row = k // (CHUNK // LANES)
        col = (k % (CHUNK // LANES)) * LANES
        pos0 = k * LANES
        fields = (pos0 + lax.iota(jnp.int32, LANES)) % N_FIELDS
        cur = idx_v[row, pl.ds(col, LANES)]
        idx_v[row, pl.ds(col, LANES)] = cur + fields * VOCAB
        return carry

    lax.fori_loop(0, N_VEC, off_body, 0)

    copies = []
    for c in range(N_CHUNKS):
        copies.append(
            pltpu.async_copy(
                w_hbm.at[idx_v.at[c]], rows_v.at[pl.ds(c * CHUNK, CHUNK)], sem
            )
        )
    for cp in copies:
        cp.wait()

    def add_body(b, carry):
        p0 = pe_v[b, pl.ds(0, LANES)]
        p1 = pe_v[b, pl.ds(LANES, LANES)]
        r0 = b * N_FIELDS
        for i in range(N_FIELDS):
            plsc.addupdate(rows_v.at[r0 + i, pl.ds(0, LANES)], p0)
            plsc.addupdate(rows_v.at[r0 + i, pl.ds(LANES, LANES)], p1)
        return carry

    lax.fori_loop(0, B_PER_W, add_body, 0)

    pltpu.sync_copy(rows_v, out_hbm.at[pl.ds(wid * ROWS_PER_W, ROWS_PER_W)])


def kernel(cat_features, W):
    cat2 = cat_features.reshape(NW, N_CHUNKS, CHUNK).astype(jnp.int32)
    w_flat = W.reshape(N_FIELDS * VOCAB, EMBED)
    out = _emb_kernel(cat2, w_flat, jnp.asarray(_PE))
    return out.reshape(BATCH, N_FIELDS * EMBED)
